# baseline (speedup 1.0000x reference)
import jax
import jax.numpy as jnp
from jax import lax
from jax.experimental import pallas as pl
from jax.experimental.pallas import tpu as pltpu
from jax.experimental.pallas import tpu_sc as plsc

N = 10000
NP = 10240
E = 160000
EP = 163840
D_IN = 256
HID = 32
HIDP = 128
G = 64
NTILES = 32
CHUNKS = EP // (NTILES * 128)
ROWS_PER_TILE = NP // 16
BLK = 2048
NBLK = NP // BLK

_mesh = plsc.VectorSubcoreMesh(core_axis_name="c", subcore_axis_name="s")


def _deg_body(dst_hbm, ones_hbm, zero_hbm, out_hbm, idx_v, ones_v, acc, sem):
    cid = lax.axis_index("c")
    sid = lax.axis_index("s")
    wid = cid * 16 + sid
    base = sid * ROWS_PER_TILE
    pltpu.sync_copy(zero_hbm, acc.at[pl.ds(base, ROWS_PER_TILE)])
    pltpu.sync_copy(ones_hbm, ones_v)
    pltpu.sync_copy(dst_hbm.at[wid], idx_v)
    plsc.subcore_barrier()

    @pl.loop(0, CHUNKS)
    def _(j):
        pltpu.sync_copy(ones_v, acc.at[idx_v.at[j]], add=True)

    plsc.subcore_barrier()
    pltpu.sync_copy(acc.at[pl.ds(base, ROWS_PER_TILE)],
                    out_hbm.at[cid, pl.ds(base, ROWS_PER_TILE)])


_deg_call = pl.kernel(
    _deg_body,
    out_type=jax.ShapeDtypeStruct((2, NP), jnp.float32),
    mesh=_mesh,
    scratch_types=[
        pltpu.VMEM((CHUNKS, 128), jnp.int32),
        pltpu.VMEM((128,), jnp.float32),
        pltpu.VMEM_SHARED((NP,), jnp.float32),
        pltpu.SemaphoreType.DMA,
    ],
)


def _edge_body(hp_hbm, src_hbm, dst_hbm, zero_hbm, out_hbm,
               sidx_v, didx_v, rows_v, acc, sem):
    cid = lax.axis_index("c")
    sid = lax.axis_index("s")
    wid = cid * 16 + sid
    base = sid * ROWS_PER_TILE
    pltpu.sync_copy(zero_hbm, acc.at[pl.ds(base, ROWS_PER_TILE)])
    pltpu.sync_copy(src_hbm.at[wid], sidx_v)
    pltpu.sync_copy(dst_hbm.at[wid], didx_v)
    plsc.subcore_barrier()

    @pl.loop(0, CHUNKS)
    def _(j):
        pltpu.async_copy(hp_hbm.at[sidx_v.at[j]], rows_v, sem).wait()
        pltpu.sync_copy(rows_v, acc.at[didx_v.at[j]], add=True)

    plsc.subcore_barrier()
    pltpu.sync_copy(acc.at[pl.ds(base, ROWS_PER_TILE)],
                    out_hbm.at[cid, pl.ds(base, ROWS_PER_TILE)])


_edge_call = pl.kernel(
    _edge_body,
    out_type=jax.ShapeDtypeStruct((2, NP, HIDP), jnp.float32),
    mesh=_mesh,
    scratch_types=[
        pltpu.VMEM((CHUNKS, 128), jnp.int32),
        pltpu.VMEM((CHUNKS, 128), jnp.int32),
        pltpu.VMEM((128, HIDP), jnp.float32),
        pltpu.VMEM_SHARED((NP, HIDP), jnp.float32),
        pltpu.SemaphoreType.DMA,
    ],
)


def _mm_scale_body(x_ref, w_ref, degp_ref, out_ref):
    deg = degp_ref[0] + degp_ref[1] + 1.0
    dinv = 1.0 / jnp.sqrt(deg)
    h = jnp.dot(x_ref[...], w_ref[...], preferred_element_type=jnp.float32,
                precision=lax.Precision.HIGHEST)
    out_ref[...] = h * dinv[:, None]


def _mm_scale(x_pad, W1p, degp):
    return pl.pallas_call(
        _mm_scale_body,
        grid=(NBLK,),
        in_specs=[
            pl.BlockSpec((BLK, D_IN), lambda i: (i, 0)),
            pl.BlockSpec((D_IN, HIDP), lambda i: (0, 0)),
            pl.BlockSpec((2, BLK), lambda i: (0, i)),
        ],
        out_specs=pl.BlockSpec((BLK, HIDP), lambda i: (i, 0)),
        out_shape=jax.ShapeDtypeStruct((NP, HIDP), jnp.float32),
    )(x_pad, W1p, degp)


def _combine_body(s_ref, hp_ref, degp_ref, b_ref, w_ref, out_ref):
    deg = degp_ref[0] + degp_ref[1] + 1.0
    dinv = 1.0 / jnp.sqrt(deg)
    s = s_ref[0] + s_ref[1] + hp_ref[...]
    h = jnp.maximum(dinv[:, None] * s + b_ref[0][None, :], 0.0)
    out_ref[...] = jnp.dot(h, w_ref[...], preferred_element_type=jnp.float32,
                           precision=lax.Precision.HIGHEST) * dinv[:, None]


def _combine_mm(s1, hp1, degp, b1b, W2p):
    return pl.pallas_call(
        _combine_body,
        grid=(NBLK,),
        in_specs=[
            pl.BlockSpec((2, BLK, HIDP), lambda i: (0, i, 0)),
            pl.BlockSpec((BLK, HIDP), lambda i: (i, 0)),
            pl.BlockSpec((2, BLK), lambda i: (0, i)),
            pl.BlockSpec((8, HIDP), lambda i: (0, 0)),
            pl.BlockSpec((HIDP, HIDP), lambda i: (0, 0)),
        ],
        out_specs=pl.BlockSpec((BLK, HIDP), lambda i: (i, 0)),
        out_shape=jax.ShapeDtypeStruct((NP, HIDP), jnp.float32),
    )(s1, hp1, degp, b1b, W2p)


def _final_body(s_ref, hp_ref, degp_ref, b_ref, batch_ref, fcwt_ref, fcbb_ref,
                out_ref, sums_acc, cnt_acc):
    i = pl.program_id(0)

    @pl.when(i == 0)
    def _():
        sums_acc[...] = jnp.zeros_like(sums_acc)
        cnt_acc[...] = jnp.zeros_like(cnt_acc)

    deg = degp_ref[0] + degp_ref[1] + 1.0
    dinv = 1.0 / jnp.sqrt(deg)
    s = s_ref[0] + s_ref[1] + hp_ref[...]
    h = jnp.maximum(dinv[:, None] * s + b_ref[0][None, :], 0.0)
    bvec = batch_ref[0, 0, :]
    onehot = (bvec[None, :] ==
              lax.broadcasted_iota(jnp.int32, (G, BLK), 0)).astype(jnp.float32)
    sums_acc[...] += jnp.dot(onehot, h, preferred_element_type=jnp.float32,
                             precision=lax.Precision.HIGHEST)
    cnt_acc[...] += jnp.sum(onehot, axis=1, keepdims=True)

    @pl.when(i == NBLK - 1)
    def _():
        g = sums_acc[...] / jnp.maximum(cnt_acc[...], 1.0)
        out_ref[...] = (jnp.sum(g * fcwt_ref[0][None, :], axis=1,
                                keepdims=True) + fcbb_ref[0, 0])


def _final(s2, hp2, degp, b2b, batch3, fcwt, fcbb):
    return pl.pallas_call(
        _final_body,
        grid=(NBLK,),
        in_specs=[
            pl.BlockSpec((2, BLK, HIDP), lambda i: (0, i, 0)),
            pl.BlockSpec((BLK, HIDP), lambda i: (i, 0)),
            pl.BlockSpec((2, BLK), lambda i: (0, i)),
            pl.BlockSpec((8, HIDP), lambda i: (0, 0)),
            pl.BlockSpec((1, 1, BLK), lambda i: (i, 0, 0)),
            pl.BlockSpec((8, HIDP), lambda i: (0, 0)),
            pl.BlockSpec((8, HIDP), lambda i: (0, 0)),
        ],
        out_specs=pl.BlockSpec((G, 1), lambda i: (0, 0)),
        out_shape=jax.ShapeDtypeStruct((G, 1), jnp.float32),
        scratch_shapes=[
            pltpu.VMEM((G, HIDP), jnp.float32),
            pltpu.VMEM((G, 1), jnp.float32),
        ],
    )(s2, hp2, degp, b2b, batch3, fcwt, fcbb)


def kernel(x, edge_index, batch, W1, b1, W2, b2, fcW, fcb):
    src = edge_index[0].astype(jnp.int32)
    dst = edge_index[1].astype(jnp.int32)
    pad_e = EP - E
    src_r = jnp.concatenate(
        [src, jnp.full((pad_e,), N, jnp.int32)]).reshape(NTILES, CHUNKS, 128)
    dst_r = jnp.concatenate(
        [dst, jnp.full((pad_e,), N, jnp.int32)]).reshape(NTILES, CHUNKS, 128)
    x_pad = jnp.pad(x, ((0, NP - N), (0, 0)))
    batch3 = jnp.concatenate(
        [batch.astype(jnp.int32), jnp.full((NP - N,), G, jnp.int32)]
    ).reshape(NBLK, 1, BLK)
    ones128 = jnp.ones((128,), jnp.float32)
    zdeg = jnp.zeros((ROWS_PER_TILE,), jnp.float32)
    zrows = jnp.zeros((ROWS_PER_TILE, HIDP), jnp.float32)
    W1p = jnp.pad(W1, ((0, 0), (0, HIDP - HID)))
    W2p = jnp.pad(W2, ((0, HIDP - HID), (0, HIDP - HID)))
    b1b = jnp.broadcast_to(jnp.pad(b1, (0, HIDP - HID))[None, :], (8, HIDP))
    b2b = jnp.broadcast_to(jnp.pad(b2, (0, HIDP - HID))[None, :], (8, HIDP))
    fcwt = jnp.broadcast_to(jnp.pad(fcW[:, 0], (0, HIDP - HID))[None, :],
                            (8, HIDP))
    fcbb = jnp.broadcast_to(jnp.pad(fcb, (0, HIDP - 1))[None, :], (8, HIDP))

    degp = _deg_call(dst_r, ones128, zdeg)
    hp1 = _mm_scale(x_pad, W1p, degp)
    s1 = _edge_call(hp1, src_r, dst_r, zrows)
    hp2 = _combine_mm(s1, hp1, degp, b1b, W2p)
    s2 = _edge_call(hp2, src_r, dst_r, zrows)
    out2d = _final(s2, hp2, degp, b2b, batch3, fcwt, fcbb)
    return out2d[:, 0]


# speedup vs baseline: 1.1171x; 1.1171x over previous
import jax
import jax.numpy as jnp
from jax import lax
from jax.experimental import pallas as pl
from jax.experimental.pallas import tpu as pltpu
from jax.experimental.pallas import tpu_sc as plsc

N = 10000
NP = 10240
E = 160000
EP = 163840
D_IN = 256
HID = 32
HIDP = 128
G = 64
NTILES = 32
CHUNKS = EP // (NTILES * 128)
ROWS_PER_TILE = NP // 16
BLK = 2048
NBLK = NP // BLK

_mesh = plsc.VectorSubcoreMesh(core_axis_name="c", subcore_axis_name="s")


def _deg_body(dst_hbm, ones_hbm, zero_hbm, out_hbm, idx_v, ones_v, acc, sem):
    cid = lax.axis_index("c")
    sid = lax.axis_index("s")
    wid = cid * 16 + sid
    base = sid * ROWS_PER_TILE
    pltpu.sync_copy(zero_hbm, acc.at[pl.ds(base, ROWS_PER_TILE)])
    pltpu.sync_copy(ones_hbm, ones_v)
    pltpu.sync_copy(dst_hbm.at[wid], idx_v)
    plsc.subcore_barrier()

    @pl.loop(0, CHUNKS)
    def _(j):
        pltpu.sync_copy(ones_v, acc.at[idx_v.at[j]], add=True)

    plsc.subcore_barrier()
    pltpu.sync_copy(acc.at[pl.ds(base, ROWS_PER_TILE)],
                    out_hbm.at[cid, pl.ds(base, ROWS_PER_TILE)])


_deg_call = pl.kernel(
    _deg_body,
    out_type=jax.ShapeDtypeStruct((2, NP), jnp.float32),
    mesh=_mesh,
    scratch_types=[
        pltpu.VMEM((CHUNKS, 128), jnp.int32),
        pltpu.VMEM((128,), jnp.float32),
        pltpu.VMEM_SHARED((NP,), jnp.float32),
        pltpu.SemaphoreType.DMA,
    ],
)


NBUF = 2


def _edge_body(hp_hbm, src_hbm, dst_hbm, zero_hbm, out_hbm,
               sidx_v, didx_v, r0, r1, acc, s0, s1):
    cid = lax.axis_index("c")
    sid = lax.axis_index("s")
    wid = cid * 16 + sid
    base = sid * ROWS_PER_TILE
    rows = [r0, r1]
    sems = [s0, s1]
    pltpu.sync_copy(zero_hbm, acc.at[pl.ds(base, ROWS_PER_TILE)])
    pltpu.sync_copy(src_hbm.at[wid], sidx_v)
    pltpu.sync_copy(dst_hbm.at[wid], didx_v)
    plsc.subcore_barrier()

    for b in range(NBUF):
        pltpu.async_copy(hp_hbm.at[sidx_v.at[b]], rows[b], sems[b])

    @pl.loop(0, CHUNKS - NBUF, step=NBUF)
    def _(g):
        for b in range(NBUF):
            j = g + b
            pltpu.make_async_copy(hp_hbm.at[sidx_v.at[j]], rows[b],
                                  sems[b]).wait()
            pltpu.sync_copy(rows[b], acc.at[didx_v.at[j]], add=True)
            pltpu.async_copy(hp_hbm.at[sidx_v.at[j + NBUF]], rows[b], sems[b])

    for b in range(NBUF):
        j = CHUNKS - NBUF + b
        pltpu.make_async_copy(hp_hbm.at[sidx_v.at[j]], rows[b], sems[b]).wait()
        pltpu.sync_copy(rows[b], acc.at[didx_v.at[j]], add=True)

    plsc.subcore_barrier()
    pltpu.sync_copy(acc.at[pl.ds(base, ROWS_PER_TILE)],
                    out_hbm.at[cid, pl.ds(base, ROWS_PER_TILE)])


_edge_call = pl.kernel(
    _edge_body,
    out_type=jax.ShapeDtypeStruct((2, NP, HIDP), jnp.float32),
    mesh=_mesh,
    scratch_types=[
        pltpu.VMEM((CHUNKS, 128), jnp.int32),
        pltpu.VMEM((CHUNKS, 128), jnp.int32),
        pltpu.VMEM((128, HIDP), jnp.float32),
        pltpu.VMEM((128, HIDP), jnp.float32),
        pltpu.VMEM_SHARED((NP, HIDP), jnp.float32),
        pltpu.SemaphoreType.DMA,
        pltpu.SemaphoreType.DMA,
    ],
)


def _mm_scale_body(x_ref, w_ref, degp_ref, out_ref):
    deg = degp_ref[0] + degp_ref[1] + 1.0
    dinv = 1.0 / jnp.sqrt(deg)
    h = jnp.dot(x_ref[...], w_ref[...], preferred_element_type=jnp.float32,
                precision=lax.Precision.HIGHEST)
    out_ref[...] = h * dinv[:, None]


def _mm_scale(x_pad, W1p, degp):
    return pl.pallas_call(
        _mm_scale_body,
        grid=(NBLK,),
        in_specs=[
            pl.BlockSpec((BLK, D_IN), lambda i: (i, 0)),
            pl.BlockSpec((D_IN, HIDP), lambda i: (0, 0)),
            pl.BlockSpec((2, BLK), lambda i: (0, i)),
        ],
        out_specs=pl.BlockSpec((BLK, HIDP), lambda i: (i, 0)),
        out_shape=jax.ShapeDtypeStruct((NP, HIDP), jnp.float32),
    )(x_pad, W1p, degp)


def _combine_body(s_ref, hp_ref, degp_ref, b_ref, w_ref, out_ref):
    deg = degp_ref[0] + degp_ref[1] + 1.0
    dinv = 1.0 / jnp.sqrt(deg)
    s = s_ref[0] + s_ref[1] + hp_ref[...]
    h = jnp.maximum(dinv[:, None] * s + b_ref[0][None, :], 0.0)
    out_ref[...] = jnp.dot(h, w_ref[...], preferred_element_type=jnp.float32,
                           precision=lax.Precision.HIGHEST) * dinv[:, None]


def _combine_mm(s1, hp1, degp, b1b, W2p):
    return pl.pallas_call(
        _combine_body,
        grid=(NBLK,),
        in_specs=[
            pl.BlockSpec((2, BLK, HIDP), lambda i: (0, i, 0)),
            pl.BlockSpec((BLK, HIDP), lambda i: (i, 0)),
            pl.BlockSpec((2, BLK), lambda i: (0, i)),
            pl.BlockSpec((8, HIDP), lambda i: (0, 0)),
            pl.BlockSpec((HIDP, HIDP), lambda i: (0, 0)),
        ],
        out_specs=pl.BlockSpec((BLK, HIDP), lambda i: (i, 0)),
        out_shape=jax.ShapeDtypeStruct((NP, HIDP), jnp.float32),
    )(s1, hp1, degp, b1b, W2p)


def _final_body(s_ref, hp_ref, degp_ref, b_ref, batch_ref, fcwt_ref, fcbb_ref,
                out_ref, sums_acc, cnt_acc):
    i = pl.program_id(0)

    @pl.when(i == 0)
    def _():
        sums_acc[...] = jnp.zeros_like(sums_acc)
        cnt_acc[...] = jnp.zeros_like(cnt_acc)

    deg = degp_ref[0] + degp_ref[1] + 1.0
    dinv = 1.0 / jnp.sqrt(deg)
    s = s_ref[0] + s_ref[1] + hp_ref[...]
    h = jnp.maximum(dinv[:, None] * s + b_ref[0][None, :], 0.0)
    bvec = batch_ref[0, 0, :]
    onehot = (bvec[None, :] ==
              lax.broadcasted_iota(jnp.int32, (G, BLK), 0)).astype(jnp.float32)
    sums_acc[...] += jnp.dot(onehot, h, preferred_element_type=jnp.float32,
                             precision=lax.Precision.HIGHEST)
    cnt_acc[...] += jnp.sum(onehot, axis=1, keepdims=True)

    @pl.when(i == NBLK - 1)
    def _():
        g = sums_acc[...] / jnp.maximum(cnt_acc[...], 1.0)
        out_ref[...] = (jnp.sum(g * fcwt_ref[0][None, :], axis=1,
                                keepdims=True) + fcbb_ref[0, 0])


def _final(s2, hp2, degp, b2b, batch3, fcwt, fcbb):
    return pl.pallas_call(
        _final_body,
        grid=(NBLK,),
        in_specs=[
            pl.BlockSpec((2, BLK, HIDP), lambda i: (0, i, 0)),
            pl.BlockSpec((BLK, HIDP), lambda i: (i, 0)),
            pl.BlockSpec((2, BLK), lambda i: (0, i)),
            pl.BlockSpec((8, HIDP), lambda i: (0, 0)),
            pl.BlockSpec((1, 1, BLK), lambda i: (i, 0, 0)),
            pl.BlockSpec((8, HIDP), lambda i: (0, 0)),
            pl.BlockSpec((8, HIDP), lambda i: (0, 0)),
        ],
        out_specs=pl.BlockSpec((G, 1), lambda i: (0, 0)),
        out_shape=jax.ShapeDtypeStruct((G, 1), jnp.float32),
        scratch_shapes=[
            pltpu.VMEM((G, HIDP), jnp.float32),
            pltpu.VMEM((G, 1), jnp.float32),
        ],
    )(s2, hp2, degp, b2b, batch3, fcwt, fcbb)


def kernel(x, edge_index, batch, W1, b1, W2, b2, fcW, fcb):
    src = edge_index[0].astype(jnp.int32)
    dst = edge_index[1].astype(jnp.int32)
    pad_e = EP - E
    src_r = jnp.concatenate(
        [src, jnp.full((pad_e,), N, jnp.int32)]).reshape(NTILES, CHUNKS, 128)
    dst_r = jnp.concatenate(
        [dst, jnp.full((pad_e,), N, jnp.int32)]).reshape(NTILES, CHUNKS, 128)
    x_pad = jnp.pad(x, ((0, NP - N), (0, 0)))
    batch3 = jnp.concatenate(
        [batch.astype(jnp.int32), jnp.full((NP - N,), G, jnp.int32)]
    ).reshape(NBLK, 1, BLK)
    ones128 = jnp.ones((128,), jnp.float32)
    zdeg = jnp.zeros((ROWS_PER_TILE,), jnp.float32)
    zrows = jnp.zeros((ROWS_PER_TILE, HIDP), jnp.float32)
    W1p = jnp.pad(W1, ((0, 0), (0, HIDP - HID)))
    W2p = jnp.pad(W2, ((0, HIDP - HID), (0, HIDP - HID)))
    b1b = jnp.broadcast_to(jnp.pad(b1, (0, HIDP - HID))[None, :], (8, HIDP))
    b2b = jnp.broadcast_to(jnp.pad(b2, (0, HIDP - HID))[None, :], (8, HIDP))
    fcwt = jnp.broadcast_to(jnp.pad(fcW[:, 0], (0, HIDP - HID))[None, :],
                            (8, HIDP))
    fcbb = jnp.broadcast_to(jnp.pad(fcb, (0, HIDP - 1))[None, :], (8, HIDP))

    degp = _deg_call(dst_r, ones128, zdeg)
    hp1 = _mm_scale(x_pad, W1p, degp)
    s1 = _edge_call(hp1, src_r, dst_r, zrows)
    hp2 = _combine_mm(s1, hp1, degp, b1b, W2p)
    s2 = _edge_call(hp2, src_r, dst_r, zrows)
    out2d = _final(s2, hp2, degp, b2b, batch3, fcwt, fcbb)
    return out2d[:, 0]
